# baseline (speedup 1.0000x reference)
import functools

import jax
import jax.numpy as jnp
from jax import lax
from jax.experimental import pallas as pl
from jax.experimental.pallas import tpu as pltpu
from jax.experimental.pallas import tpu_sc as plsc

N = 10000
E = 320000
IN_DIM = 128
HID = 32
NCLS = 2

NC = 2
NS = 16
NW = NC * NS
CHUNK = 128
GK = 4
NCH = 80
E_PAD = NW * NCH * CHUNK
NP = 10112
RPT = NP // NS
DEG_W = 16
DEG_KB = 8

_mesh = plsc.VectorSubcoreMesh(core_axis_name="c", subcore_axis_name="s")
_sc_params = pltpu.CompilerParams(use_tc_tiling_on_sc=False)


def _worker_id():
    return lax.axis_index("s") * NC + lax.axis_index("c")


@functools.partial(
    pl.kernel,
    out_type=jax.ShapeDtypeStruct((NC, NP, DEG_W), jnp.float32),
    mesh=_mesh,
    compiler_params=_sc_params,
    scratch_types=[
        pltpu.VMEM_SHARED((NP, DEG_W), jnp.float32),
        pltpu.VMEM((CHUNK, DEG_W), jnp.float32),
        pltpu.VMEM((NCH, CHUNK), jnp.int32),
        pltpu.SemaphoreType.DMA,
    ],
)
def _sc_degree(dst3, ones_hbm, zeros_hbm, out, acc, ones_v, idx_d, sem):
    cid = lax.axis_index("c")
    sid = lax.axis_index("s")
    wid = _worker_id()
    base = sid * RPT
    pltpu.sync_copy(dst3.at[wid], idx_d)
    pltpu.sync_copy(ones_hbm, ones_v)
    pltpu.sync_copy(zeros_hbm.at[pl.ds(base, RPT)], acc.at[pl.ds(base, RPT)])
    plsc.subcore_barrier()

    def body(j0, carry):
        descs = [
            pltpu.async_copy(ones_v, acc.at[idx_d.at[j0 * DEG_KB + b]], sem,
                             add=True)
            for b in range(DEG_KB)
        ]
        for d in descs:
            d.wait()
        return carry

    lax.fori_loop(0, NCH // DEG_KB, body, 0)
    plsc.subcore_barrier()
    pltpu.sync_copy(acc.at[pl.ds(base, RPT)], out.at[cid, pl.ds(base, RPT)])


def _make_spmm(dim, gk, nhalf):
    hch = NCH // nhalf

    @functools.partial(
        pl.kernel,
        out_type=jax.ShapeDtypeStruct((NC, NP, dim), jnp.float32),
        mesh=_mesh,
        compiler_params=_sc_params,
        scratch_types=[
            pltpu.VMEM_SHARED((NP, dim), jnp.float32),
            [pltpu.VMEM((CHUNK, dim), jnp.float32) for _ in range(2 * gk)],
            pltpu.VMEM((hch, CHUNK), jnp.int32),
            pltpu.VMEM((hch, CHUNK), jnp.int32),
            pltpu.SemaphoreType.DMA,
            pltpu.SemaphoreType.DMA,
        ],
    )
    def _sc_spmm(xp, src3, dst3, zeros_hbm, out, acc, rows, idx_s, idx_d,
                 gsem, ssem):
        cid = lax.axis_index("c")
        sid = lax.axis_index("s")
        wid = _worker_id()
        base = sid * RPT
        pltpu.sync_copy(zeros_hbm.at[pl.ds(base, RPT)],
                        acc.at[pl.ds(base, RPT)])
        plsc.subcore_barrier()

        def body(j0, carry):
            c0 = j0 * 2 * gk
            ga = [pltpu.async_copy(xp.at[idx_s.at[c0 + b]], rows[b], gsem)
                  for b in range(gk)]
            for d in ga:
                d.wait()
            sa = [pltpu.async_copy(rows[b], acc.at[idx_d.at[c0 + b]], ssem,
                                   add=True) for b in range(gk)]
            gb = [pltpu.async_copy(xp.at[idx_s.at[c0 + gk + b]],
                                   rows[gk + b], gsem) for b in range(gk)]
            for d in gb:
                d.wait()
            sb = [pltpu.async_copy(rows[gk + b],
                                   acc.at[idx_d.at[c0 + gk + b]],
                                   ssem, add=True) for b in range(gk)]
            for d in sa + sb:
                d.wait()
            return carry

        for h in range(nhalf):
            pltpu.sync_copy(src3.at[wid, pl.ds(h * hch, hch)], idx_s)
            pltpu.sync_copy(dst3.at[wid, pl.ds(h * hch, hch)], idx_d)
            lax.fori_loop(0, hch // (2 * gk), body, 0)
        plsc.subcore_barrier()
        pltpu.sync_copy(acc.at[pl.ds(base, RPT)], out.at[cid, pl.ds(base, RPT)])

    return _sc_spmm


_sc_spmm32 = _make_spmm(HID, GK, 1)
_sc_spmm128 = _make_spmm(IN_DIM, 1, 2)


@functools.partial(
    pl.kernel,
    out_type=(
        jax.ShapeDtypeStruct((E_PAD, HID), jnp.float32),
        jax.ShapeDtypeStruct((E_PAD, HID), jnp.float32),
    ),
    mesh=_mesh,
    compiler_params=_sc_params,
    scratch_types=[
        [pltpu.VMEM((CHUNK, HID), jnp.float32) for _ in range(2 * GK)],
        [pltpu.VMEM((CHUNK, HID), jnp.float32) for _ in range(2 * GK)],
        pltpu.VMEM((NCH, CHUNK), jnp.int32),
        pltpu.VMEM((NCH, CHUNK), jnp.int32),
        pltpu.SemaphoreType.DMA,
        pltpu.SemaphoreType.DMA,
    ],
)
def _sc_edge_gather(h3, src3, dst3, gs, gd, rows_s, rows_d, idx_s, idx_d,
                    gsem, wsem):
    wid = _worker_id()
    woff = wid * (NCH * CHUNK)
    pltpu.sync_copy(src3.at[wid], idx_s)
    pltpu.sync_copy(dst3.at[wid], idx_d)

    def grp_gather(c0, lo):
        descs = []
        for b in range(GK):
            descs.append(pltpu.async_copy(h3.at[idx_s.at[c0 + b]],
                                          rows_s[lo + b], gsem))
            descs.append(pltpu.async_copy(h3.at[idx_d.at[c0 + b]],
                                          rows_d[lo + b], gsem))
        return descs

    def grp_write(c0, lo):
        descs = []
        for b in range(GK):
            j = c0 + b
            descs.append(pltpu.async_copy(
                rows_s[lo + b], gs.at[pl.ds(woff + j * CHUNK, CHUNK)], wsem))
            descs.append(pltpu.async_copy(
                rows_d[lo + b], gd.at[pl.ds(woff + j * CHUNK, CHUNK)], wsem))
        return descs

    def body(j0, carry):
        c0 = j0 * 2 * GK
        ga = grp_gather(c0, 0)
        for d in ga:
            d.wait()
        wa = grp_write(c0, 0)
        gb = grp_gather(c0 + GK, GK)
        for d in gb:
            d.wait()
        wb = grp_write(c0 + GK, GK)
        for d in wa + wb:
            d.wait()
        return carry

    lax.fori_loop(0, NCH // (2 * GK), body, 0)


_BLK = 2528
_EPS = 1e-5


def _prep_body(nf, degp, xp, dv):
    deg = degp[0][:, :1] + degp[1][:, :1] + 1.0
    di = lax.rsqrt(deg)
    xp[...] = di * nf[...]
    dv[...] = di


def _tc_prep(nf_p, degp):
    return pl.pallas_call(
        _prep_body,
        grid=(NP // _BLK,),
        in_specs=[
            pl.BlockSpec((_BLK, IN_DIM), lambda i: (i, 0)),
            pl.BlockSpec((NC, _BLK, DEG_W), lambda i: (0, i, 0)),
        ],
        out_specs=[
            pl.BlockSpec((_BLK, IN_DIM), lambda i: (i, 0)),
            pl.BlockSpec((_BLK, 1), lambda i: (i, 0)),
        ],
        out_shape=[
            jax.ShapeDtypeStruct((NP, IN_DIM), jnp.float32),
            jax.ShapeDtypeStruct((NP, 1), jnp.float32),
        ],
    )(nf_p, degp)


def _layer_body(z, tp, dv, wb, b, g, be, out, *, scale_out):
    di = dv[...]
    zz = di * (z[0] + z[1] + tp[...])
    s = jnp.dot(zz.astype(jnp.bfloat16), wb[...],
                preferred_element_type=jnp.float32) + b[...]
    mu = jnp.mean(s, axis=-1, keepdims=True)
    var = jnp.mean((s - mu) ** 2, axis=-1, keepdims=True)
    h = jnp.maximum((s - mu) / jnp.sqrt(var + _EPS) * g[...] + be[...], 0.0)
    out[...] = di * h if scale_out else h


def _tc_layer(z, tp, dv, wb, b, g, be, scale_out):
    dim = wb.shape[0]
    vec_spec = pl.BlockSpec((1, HID), lambda i: (0, 0))
    in_specs = [pl.BlockSpec((NC, _BLK, dim), lambda i: (0, i, 0)),
                pl.BlockSpec((_BLK, dim), lambda i: (i, 0)),
                pl.BlockSpec((_BLK, 1), lambda i: (i, 0)),
                pl.BlockSpec((dim, HID), lambda i: (0, 0)),
                vec_spec, vec_spec, vec_spec]
    body = functools.partial(_layer_body, scale_out=scale_out)
    return pl.pallas_call(
        body,
        grid=(NP // _BLK,),
        in_specs=in_specs,
        out_specs=pl.BlockSpec((_BLK, HID), lambda i: (i, 0)),
        out_shape=jax.ShapeDtypeStruct((NP, HID), jnp.float32),
    )(z, tp, dv, wb, b, g, be)


_EBLK = 640


def _head_body(gs, gd, w4, b4, out):
    p = (gs[...] * gd[...]).astype(jnp.bfloat16)
    out[...] = (jnp.dot(p, w4[...], preferred_element_type=jnp.float32)
                + b4[...])


def _tc_head(gs4, gd4, w4, b4):
    return pl.pallas_call(
        _head_body,
        grid=(E // (4 * _EBLK),),
        in_specs=[
            pl.BlockSpec((_EBLK, 128), lambda i: (i, 0)),
            pl.BlockSpec((_EBLK, 128), lambda i: (i, 0)),
            pl.BlockSpec((128, 4 * NCLS), lambda i: (0, 0)),
            pl.BlockSpec((1, 4 * NCLS), lambda i: (0, 0)),
        ],
        out_specs=pl.BlockSpec((_EBLK, 4 * NCLS), lambda i: (i, 0)),
        out_shape=jax.ShapeDtypeStruct((E // 4, 4 * NCLS), jnp.float32),
    )(gs4, gd4, w4, b4)


def kernel(node_feat, edge_index, fc1_w, fc1_b, fc2_w, fc2_b, fc3_w, fc3_b,
           ln1_g, ln1_b, ln2_g, ln2_b, ln3_g, ln3_b, cls_w, cls_b):
    ei = edge_index.astype(jnp.int32)
    src = jnp.concatenate(
        [ei[0], jnp.zeros((E_PAD - E,), jnp.int32)]).reshape(NW, NCH, CHUNK)
    dst = jnp.concatenate(
        [ei[1], jnp.full((E_PAD - E,), N, jnp.int32)]).reshape(NW, NCH, CHUNK)

    nf_p = jnp.pad(node_feat, ((0, NP - N), (0, 0)))
    zeros_deg = jnp.zeros((NP, DEG_W), jnp.float32)
    ones_deg = jnp.ones((CHUNK, DEG_W), jnp.float32)
    zeros_hid = jnp.zeros((NP, HID), jnp.float32)
    zeros_in = jnp.zeros((NP, IN_DIM), jnp.float32)

    degp = _sc_degree(dst, ones_deg, zeros_deg)

    xp, dv = _tc_prep(nf_p, degp)
    z = _sc_spmm128(xp, src, dst, zeros_in)
    xp = _tc_layer(z, xp, dv, fc1_w.T.astype(jnp.bfloat16),
                   fc1_b.reshape(1, HID), ln1_g.reshape(1, HID),
                   ln1_b.reshape(1, HID), True)
    z = _sc_spmm32(xp, src, dst, zeros_hid)
    xp = _tc_layer(z, xp, dv, fc2_w.T.astype(jnp.bfloat16),
                   fc2_b.reshape(1, HID), ln2_g.reshape(1, HID),
                   ln2_b.reshape(1, HID), True)
    z = _sc_spmm32(xp, src, dst, zeros_hid)
    h3 = _tc_layer(z, xp, dv, fc3_w.T.astype(jnp.bfloat16),
                   fc3_b.reshape(1, HID), ln3_g.reshape(1, HID),
                   ln3_b.reshape(1, HID), False)

    gs, gd = _sc_edge_gather(h3, src, dst)
    w4 = jnp.kron(jnp.eye(4, dtype=jnp.float32),
                  cls_w.T).astype(jnp.bfloat16)
    b4 = jnp.tile(cls_b, 4).reshape(1, 4 * NCLS)
    out = _tc_head(gs.reshape(E_PAD // 4, 128), gd.reshape(E_PAD // 4, 128),
                   w4, b4)
    return out.reshape(E, NCLS)
